# baseline (speedup 1.0000x reference)
import functools

import jax
import jax.numpy as jnp
from jax import lax
from jax.experimental import pallas as pl
from jax.experimental.pallas import tpu as pltpu
from jax.experimental.pallas import tpu_sc as plsc

B = 16384
D = 16
EPS = 1e-3

_NC = 2
_NS = 16
_NW = _NC * _NS
_BPW = B // _NW
_G = 16
_NGRP = _BPW // _G

_MESH = plsc.VectorSubcoreMesh(core_axis_name="c", subcore_axis_name="s")


def _sc_gather(user, item, ucfT, icfT, ub1d, ib1d):
    @functools.partial(
        pl.kernel,
        out_type=(
            jax.ShapeDtypeStruct((D, B), jnp.float32),
            jax.ShapeDtypeStruct((D, B), jnp.float32),
            jax.ShapeDtypeStruct((B,), jnp.float32),
            jax.ShapeDtypeStruct((B,), jnp.float32),
        ),
        mesh=_MESH,
        compiler_params=pltpu.CompilerParams(needs_layout_passes=False),
        scratch_types=[
            pltpu.VMEM((_BPW,), jnp.int32),
            pltpu.VMEM((_BPW,), jnp.int32),
            pltpu.VMEM((D, _G * 128), jnp.float32),
            pltpu.VMEM((D, _G * 128), jnp.float32),
            pltpu.VMEM((D, _BPW), jnp.float32),
            pltpu.VMEM((D, _BPW), jnp.float32),
            pltpu.VMEM((_BPW,), jnp.float32),
            pltpu.VMEM((_BPW,), jnp.float32),
            pltpu.SemaphoreType.DMA,
            pltpu.SemaphoreType.DMA,
            pltpu.SemaphoreType.DMA,
        ],
    )
    def k(user_h, item_h, ucfT_h, icfT_h, ub_h, ib_h,
          u_out, i_out, bu_out, bi_out,
          uidx_v, iidx_v, bankA, bankB, u_v, i_v, bu_v, bi_v,
          semA, semB, semC):
        wid = lax.axis_index("s") * _NC + lax.axis_index("c")
        base = wid * _BPW
        pltpu.sync_copy(user_h.at[pl.ds(base, _BPW)], uidx_v)
        pltpu.sync_copy(item_h.at[pl.ds(base, _BPW)], iidx_v)
        hb0 = pltpu.async_copy(ub_h.at[uidx_v], bu_v, semC)
        hb1 = pltpu.async_copy(ib_h.at[iidx_v], bi_v, semC)
        iota16 = lax.iota(jnp.int32, 16)

        def run_table(tab_h, idx_v, cols_v):
            def fire(g, bank, sem):
                vec = idx_v[pl.ds(g * _G, _G)]
                start_v = (vec >> 7) << 7
                for l in range(_G):
                    s = pl.multiple_of(start_v[l], 128)
                    pltpu.async_copy(
                        tab_h.at[:, pl.ds(s, 128)],
                        bank.at[:, pl.ds(l * 128, 128)],
                        sem)

            def drain(bank, sem):
                pltpu.make_async_copy(
                    tab_h.at[:, pl.ds(0, _G * 128)], bank, sem).wait()

            def extract(g, bank):
                vec = idx_v[pl.ds(g * _G, _G)]
                cols16 = (vec & 127) + iota16 * 128
                for f in range(D):
                    vals = plsc.load_gather(
                        bank, [jnp.full((16,), f, jnp.int32), cols16])
                    cols_v[f, pl.ds(g * _G, _G)] = vals

            fire(0, bankA, semA)

            def step(h, _):
                fire(2 * h + 1, bankB, semB)
                drain(bankA, semA)
                extract(2 * h, bankA)

                @pl.when(h < _NGRP // 2 - 1)
                def _():
                    fire(2 * h + 2, bankA, semA)

                drain(bankB, semB)
                extract(2 * h + 1, bankB)
                return ()

            lax.fori_loop(0, _NGRP // 2, step, ())

        run_table(ucfT_h, uidx_v, u_v)
        run_table(icfT_h, iidx_v, i_v)
        pltpu.sync_copy(u_v, u_out.at[:, pl.ds(base, _BPW)])
        pltpu.sync_copy(i_v, i_out.at[:, pl.ds(base, _BPW)])
        hb0.wait()
        hb1.wait()
        pltpu.sync_copy(bu_v, bu_out.at[pl.ds(base, _BPW)])
        pltpu.sync_copy(bi_v, bi_out.at[pl.ds(base, _BPW)])

    return k(user, item, ucfT, icfT, ub1d, ib1d)


_BM = 2048


def _mlp_body(u_ref, i_ref, bu_ref, bi_ref,
              w0aT_ref, w0bT_ref, w0cT_ref, b0_ref,
              w1T_ref, b1_ref, w2T_ref, b2_ref, woT_ref, bo_ref,
              out_ref):
    u = u_ref[...]
    it = i_ref[...]
    f32 = jnp.float32
    h = (jnp.dot(w0aT_ref[...], u, preferred_element_type=f32)
         + jnp.dot(w0bT_ref[...], it, preferred_element_type=f32)
         + jnp.dot(w0cT_ref[...], u * it, preferred_element_type=f32)
         + b0_ref[...])
    h = jnp.maximum(h, 0.0)
    h = jnp.maximum(jnp.dot(w1T_ref[...], h, preferred_element_type=f32)
                    + b1_ref[...], 0.0)
    h = jnp.maximum(jnp.dot(w2T_ref[...], h, preferred_element_type=f32)
                    + b2_ref[...], 0.0)
    out_ref[...] = (jnp.dot(woT_ref[...], h, preferred_element_type=f32)
                    + bo_ref[...] + bu_ref[...] + bi_ref[...])


def _tc_mlp(uT, itT, buR, biR,
            w0aT, w0bT, w0cT, b0c, w1T, b1c, w2T, b2c, woT, boc):
    grid = (B // _BM,)
    col_spec = pl.BlockSpec((D, _BM), lambda i: (0, i))
    row_spec = pl.BlockSpec((1, _BM), lambda i: (0, i))

    def full(a):
        return pl.BlockSpec(a.shape, lambda i: tuple(0 for _ in a.shape))

    return pl.pallas_call(
        _mlp_body,
        grid=grid,
        in_specs=[col_spec, col_spec, row_spec, row_spec,
                  full(w0aT), full(w0bT), full(w0cT), full(b0c),
                  full(w1T), full(b1c), full(w2T), full(b2c),
                  full(woT), full(boc)],
        out_specs=row_spec,
        out_shape=jax.ShapeDtypeStruct((1, B), jnp.float32),
    )(uT, itT, buR, biR,
      w0aT, w0bT, w0cT, b0c, w1T, b1c, w2T, b2c, woT, boc)


def kernel(user, item, user_cf_table, item_cf_table, user_bias_table,
           item_bias_table,
           W0, b0, gamma0, beta0, mm0, mv0,
           W1, b1, gamma1, beta1, mm1, mv1,
           W2, b2, gamma2, beta2, mm2, mv2,
           W_out, b_out):
    s0 = gamma0 * lax.rsqrt(mv0 + EPS)
    t0 = beta0 - mm0 * s0
    s1 = gamma1 * lax.rsqrt(mv1 + EPS)
    t1 = beta1 - mm1 * s1
    s2 = gamma2 * lax.rsqrt(mv2 + EPS)
    t2 = beta2 - mm2 * s2

    w1f = s0[:, None] * W1
    b1f = b1 + t0 @ W1
    w2f = s1[:, None] * W2
    b2f = b2 + t1 @ W2
    wof = s2[:, None] * W_out
    bof = b_out + t2 @ W_out

    uT, itT, bu, bi = _sc_gather(user, item,
                                 user_cf_table.T, item_cf_table.T,
                                 user_bias_table.reshape(-1),
                                 item_bias_table.reshape(-1))
    predT = _tc_mlp(uT, itT, bu.reshape(1, B), bi.reshape(1, B),
                    W0[:D].T, W0[D:2 * D].T, W0[2 * D:].T, b0[:, None],
                    w1f.T, b1f[:, None], w2f.T, b2f[:, None],
                    wof.T, bof[:, None])
    return predT.reshape(B, 1)


# speedup vs baseline: 1.0027x; 1.0027x over previous
import functools

import jax
import jax.numpy as jnp
from jax import lax
from jax.experimental import pallas as pl
from jax.experimental.pallas import tpu as pltpu
from jax.experimental.pallas import tpu_sc as plsc

B = 16384
D = 16
EPS = 1e-3

_NC = 2
_NS = 16
_NW = _NC * _NS
_BPW = B // _NW
_G = 16
_NGRP = _BPW // _G

_MESH = plsc.VectorSubcoreMesh(core_axis_name="c", subcore_axis_name="s")


def _sc_gather(user, item, ucfT, icfT, ub1d, ib1d):
    @functools.partial(
        pl.kernel,
        out_type=(
            jax.ShapeDtypeStruct((D, B), jnp.float32),
            jax.ShapeDtypeStruct((D, B), jnp.float32),
            jax.ShapeDtypeStruct((B,), jnp.float32),
            jax.ShapeDtypeStruct((B,), jnp.float32),
        ),
        mesh=_MESH,
        compiler_params=pltpu.CompilerParams(needs_layout_passes=False),
        scratch_types=[
            pltpu.VMEM((_BPW,), jnp.int32),
            pltpu.VMEM((_BPW,), jnp.int32),
            pltpu.VMEM((D, _G * 128), jnp.float32),
            pltpu.VMEM((D, _G * 128), jnp.float32),
            pltpu.VMEM((D, _BPW), jnp.float32),
            pltpu.VMEM((D, _BPW), jnp.float32),
            pltpu.VMEM((_BPW,), jnp.float32),
            pltpu.VMEM((_BPW,), jnp.float32),
            pltpu.SemaphoreType.DMA,
            pltpu.SemaphoreType.DMA,
            pltpu.SemaphoreType.DMA,
        ],
    )
    def k(user_h, item_h, ucfT_h, icfT_h, ub_h, ib_h,
          u_out, i_out, bu_out, bi_out,
          uidx_v, iidx_v, bankA, bankB, u_v, i_v, bu_v, bi_v,
          semA, semB, semC):
        wid = lax.axis_index("s") * _NC + lax.axis_index("c")
        base = wid * _BPW
        pltpu.sync_copy(user_h.at[pl.ds(base, _BPW)], uidx_v)
        pltpu.sync_copy(item_h.at[pl.ds(base, _BPW)], iidx_v)
        hb0 = pltpu.async_copy(ub_h.at[uidx_v], bu_v, semC)
        hb1 = pltpu.async_copy(ib_h.at[iidx_v], bi_v, semC)
        iota16 = lax.iota(jnp.int32, 16)

        def run_table(tab_h, idx_v, cols_v):
            def fire(g, bank, sem):
                vec = idx_v[pl.ds(g * _G, _G)]
                start_v = (vec >> 7) << 7
                for l in range(_G):
                    s = pl.multiple_of(start_v[l], 128)
                    pltpu.async_copy(
                        tab_h.at[:, pl.ds(s, 128)],
                        bank.at[:, pl.ds(l * 128, 128)],
                        sem)

            def drain(bank, sem):
                pltpu.make_async_copy(
                    tab_h.at[:, pl.ds(0, _G * 128)], bank, sem).wait()

            def extract(g, bank):
                vec = idx_v[pl.ds(g * _G, _G)]
                cols16 = (vec & 127) + iota16 * 128
                for f in range(D):
                    vals = plsc.load_gather(
                        bank, [jnp.full((16,), f, jnp.int32), cols16])
                    cols_v[f, pl.ds(g * _G, _G)] = vals

            fire(0, bankA, semA)

            def step(h, _):
                fire(2 * h + 1, bankB, semB)
                drain(bankA, semA)
                extract(2 * h, bankA)

                @pl.when(h < _NGRP // 2 - 1)
                def _():
                    fire(2 * h + 2, bankA, semA)

                drain(bankB, semB)
                extract(2 * h + 1, bankB)
                return ()

            lax.fori_loop(0, _NGRP // 2, step, ())

        run_table(ucfT_h, uidx_v, u_v)
        run_table(icfT_h, iidx_v, i_v)
        pltpu.sync_copy(u_v, u_out.at[:, pl.ds(base, _BPW)])
        pltpu.sync_copy(i_v, i_out.at[:, pl.ds(base, _BPW)])
        hb0.wait()
        hb1.wait()
        pltpu.sync_copy(bu_v, bu_out.at[pl.ds(base, _BPW)])
        pltpu.sync_copy(bi_v, bi_out.at[pl.ds(base, _BPW)])

    return k(user, item, ucfT, icfT, ub1d, ib1d)


_BM = 2048


def _mlp_body(u_ref, i_ref, bu_ref, bi_ref,
              w0aT_ref, w0bT_ref, w0cT_ref, b0_ref,
              w1T_ref, b1_ref, w2T_ref, b2_ref, woT_ref, bo_ref,
              out_ref):
    u = u_ref[...]
    it = i_ref[...]
    f32 = jnp.float32
    h = (jnp.dot(w0aT_ref[...], u, preferred_element_type=f32)
         + jnp.dot(w0bT_ref[...], it, preferred_element_type=f32)
         + jnp.dot(w0cT_ref[...], u * it, preferred_element_type=f32)
         + b0_ref[...])
    h = jnp.maximum(h, 0.0)
    h = jnp.maximum(jnp.dot(w1T_ref[...], h, preferred_element_type=f32)
                    + b1_ref[...], 0.0)
    h = jnp.maximum(jnp.dot(w2T_ref[...], h, preferred_element_type=f32)
                    + b2_ref[...], 0.0)
    out_ref[...] = (jnp.dot(woT_ref[...], h, preferred_element_type=f32)
                    + bo_ref[...] + bu_ref[...] + bi_ref[...])


def _tc_mlp(uT, itT, buR, biR,
            w0aT, w0bT, w0cT, b0c, w1T, b1c, w2T, b2c, woT, boc):
    grid = (B // _BM,)
    col_spec = pl.BlockSpec((D, _BM), lambda i: (0, i))
    row_spec = pl.BlockSpec((1, _BM), lambda i: (0, i))

    def full(a):
        return pl.BlockSpec(a.shape, lambda i: tuple(0 for _ in a.shape))

    return pl.pallas_call(
        _mlp_body,
        grid=grid,
        in_specs=[col_spec, col_spec, row_spec, row_spec,
                  full(w0aT), full(w0bT), full(w0cT), full(b0c),
                  full(w1T), full(b1c), full(w2T), full(b2c),
                  full(woT), full(boc)],
        out_specs=row_spec,
        out_shape=jax.ShapeDtypeStruct((1, B), jnp.float32),
    )(uT, itT, buR, biR,
      w0aT, w0bT, w0cT, b0c, w1T, b1c, w2T, b2c, woT, boc)


def kernel(user, item, user_cf_table, item_cf_table, user_bias_table,
           item_bias_table,
           W0, b0, gamma0, beta0, mm0, mv0,
           W1, b1, gamma1, beta1, mm1, mv1,
           W2, b2, gamma2, beta2, mm2, mv2,
           W_out, b_out):
    s0 = gamma0 * lax.rsqrt(mv0 + EPS)
    t0 = beta0 - mm0 * s0
    s1 = gamma1 * lax.rsqrt(mv1 + EPS)
    t1 = beta1 - mm1 * s1
    s2 = gamma2 * lax.rsqrt(mv2 + EPS)
    t2 = beta2 - mm2 * s2

    w1f = s0[:, None] * W1
    b1f = b1 + t0 @ W1
    w2f = s1[:, None] * W2
    b2f = b2 + t1 @ W2
    wof = s2[:, None] * W_out
    bof = b_out + t2 @ W_out

    uT, itT, bu, bi = _sc_gather(user, item,
                                 user_cf_table.T, item_cf_table.T,
                                 user_bias_table[:, 0],
                                 item_bias_table[:, 0])
    predT = _tc_mlp(uT, itT, bu.reshape(1, B), bi.reshape(1, B),
                    W0[:D].T, W0[D:2 * D].T, W0[2 * D:].T, b0[:, None],
                    w1f.T, b1f[:, None], w2f.T, b2f[:, None],
                    wof.T, bof[:, None])
    return predT.reshape(B, 1)


# speedup vs baseline: 1.5850x; 1.5807x over previous
import functools

import jax
import jax.numpy as jnp
from jax import lax
from jax.experimental import pallas as pl
from jax.experimental.pallas import tpu as pltpu
from jax.experimental.pallas import tpu_sc as plsc

B = 16384
D = 16
EPS = 1e-3

_NC = 2
_NS = 16
_NW = _NC * _NS
_BPW = B // _NW
_G = 16
_NGRP = _BPW // _G

_MESH = plsc.VectorSubcoreMesh(core_axis_name="c", subcore_axis_name="s")


def _sc_gather(user, item, ucfT, icfT, ubT, ibT):
    @functools.partial(
        pl.kernel,
        out_type=(
            jax.ShapeDtypeStruct((D, B), jnp.float32),
            jax.ShapeDtypeStruct((D, B), jnp.float32),
            jax.ShapeDtypeStruct((B,), jnp.float32),
            jax.ShapeDtypeStruct((B,), jnp.float32),
        ),
        mesh=_MESH,
        compiler_params=pltpu.CompilerParams(needs_layout_passes=False),
        scratch_types=[
            pltpu.VMEM((_BPW,), jnp.int32),
            pltpu.VMEM((_BPW,), jnp.int32),
            pltpu.VMEM((D, _G * 128), jnp.float32),
            pltpu.VMEM((D, _G * 128), jnp.float32),
            pltpu.VMEM((D, _BPW), jnp.float32),
            pltpu.VMEM((D, _BPW), jnp.float32),
            pltpu.VMEM((_BPW,), jnp.float32),
            pltpu.VMEM((_BPW,), jnp.float32),
            pltpu.SemaphoreType.DMA,
            pltpu.SemaphoreType.DMA,
            pltpu.SemaphoreType.DMA,
        ],
    )
    def k(user_h, item_h, ucfT_h, icfT_h, ub_h, ib_h,
          u_out, i_out, bu_out, bi_out,
          uidx_v, iidx_v, bankA, bankB, u_v, i_v, bu_v, bi_v,
          semA, semB, semC):
        wid = lax.axis_index("s") * _NC + lax.axis_index("c")
        base = wid * _BPW
        pltpu.sync_copy(user_h.at[pl.ds(base, _BPW)], uidx_v)
        pltpu.sync_copy(item_h.at[pl.ds(base, _BPW)], iidx_v)
        hb0 = pltpu.async_copy(ub_h.at[0].at[uidx_v], bu_v, semC)
        hb1 = pltpu.async_copy(ib_h.at[0].at[iidx_v], bi_v, semC)
        iota16 = lax.iota(jnp.int32, 16)

        def run_table(tab_h, idx_v, cols_v):
            def fire(g, bank, sem):
                vec = idx_v[pl.ds(g * _G, _G)]
                start_v = (vec >> 7) << 7
                for l in range(_G):
                    s = pl.multiple_of(start_v[l], 128)
                    pltpu.async_copy(
                        tab_h.at[:, pl.ds(s, 128)],
                        bank.at[:, pl.ds(l * 128, 128)],
                        sem)

            def drain(bank, sem):
                pltpu.make_async_copy(
                    tab_h.at[:, pl.ds(0, _G * 128)], bank, sem).wait()

            def extract(g, bank):
                vec = idx_v[pl.ds(g * _G, _G)]
                cols16 = (vec & 127) + iota16 * 128
                for f in range(D):
                    vals = plsc.load_gather(
                        bank, [jnp.full((16,), f, jnp.int32), cols16])
                    cols_v[f, pl.ds(g * _G, _G)] = vals

            fire(0, bankA, semA)

            def step(h, _):
                fire(2 * h + 1, bankB, semB)
                drain(bankA, semA)
                extract(2 * h, bankA)

                @pl.when(h < _NGRP // 2 - 1)
                def _():
                    fire(2 * h + 2, bankA, semA)

                drain(bankB, semB)
                extract(2 * h + 1, bankB)
                return ()

            lax.fori_loop(0, _NGRP // 2, step, ())

        run_table(ucfT_h, uidx_v, u_v)
        run_table(icfT_h, iidx_v, i_v)
        pltpu.sync_copy(u_v, u_out.at[:, pl.ds(base, _BPW)])
        pltpu.sync_copy(i_v, i_out.at[:, pl.ds(base, _BPW)])
        hb0.wait()
        hb1.wait()
        pltpu.sync_copy(bu_v, bu_out.at[pl.ds(base, _BPW)])
        pltpu.sync_copy(bi_v, bi_out.at[pl.ds(base, _BPW)])

    return k(user, item, ucfT, icfT, ubT, ibT)


_BM = 2048


def _mlp_body(u_ref, i_ref, bu_ref, bi_ref,
              w0aT_ref, w0bT_ref, w0cT_ref, b0_ref,
              w1T_ref, b1_ref, w2T_ref, b2_ref, woT_ref, bo_ref,
              out_ref):
    u = u_ref[...]
    it = i_ref[...]
    f32 = jnp.float32
    h = (jnp.dot(w0aT_ref[...], u, preferred_element_type=f32)
         + jnp.dot(w0bT_ref[...], it, preferred_element_type=f32)
         + jnp.dot(w0cT_ref[...], u * it, preferred_element_type=f32)
         + b0_ref[...])
    h = jnp.maximum(h, 0.0)
    h = jnp.maximum(jnp.dot(w1T_ref[...], h, preferred_element_type=f32)
                    + b1_ref[...], 0.0)
    h = jnp.maximum(jnp.dot(w2T_ref[...], h, preferred_element_type=f32)
                    + b2_ref[...], 0.0)
    out_ref[...] = (jnp.dot(woT_ref[...], h, preferred_element_type=f32)
                    + bo_ref[...] + bu_ref[...] + bi_ref[...])


def _tc_mlp(uT, itT, buR, biR,
            w0aT, w0bT, w0cT, b0c, w1T, b1c, w2T, b2c, woT, boc):
    grid = (B // _BM,)
    col_spec = pl.BlockSpec((D, _BM), lambda i: (0, i))
    row_spec = pl.BlockSpec((1, _BM), lambda i: (0, i))

    def full(a):
        return pl.BlockSpec(a.shape, lambda i: tuple(0 for _ in a.shape))

    return pl.pallas_call(
        _mlp_body,
        grid=grid,
        in_specs=[col_spec, col_spec, row_spec, row_spec,
                  full(w0aT), full(w0bT), full(w0cT), full(b0c),
                  full(w1T), full(b1c), full(w2T), full(b2c),
                  full(woT), full(boc)],
        out_specs=row_spec,
        out_shape=jax.ShapeDtypeStruct((1, B), jnp.float32),
    )(uT, itT, buR, biR,
      w0aT, w0bT, w0cT, b0c, w1T, b1c, w2T, b2c, woT, boc)


def kernel(user, item, user_cf_table, item_cf_table, user_bias_table,
           item_bias_table,
           W0, b0, gamma0, beta0, mm0, mv0,
           W1, b1, gamma1, beta1, mm1, mv1,
           W2, b2, gamma2, beta2, mm2, mv2,
           W_out, b_out):
    s0 = gamma0 * lax.rsqrt(mv0 + EPS)
    t0 = beta0 - mm0 * s0
    s1 = gamma1 * lax.rsqrt(mv1 + EPS)
    t1 = beta1 - mm1 * s1
    s2 = gamma2 * lax.rsqrt(mv2 + EPS)
    t2 = beta2 - mm2 * s2

    w1f = s0[:, None] * W1
    b1f = b1 + t0 @ W1
    w2f = s1[:, None] * W2
    b2f = b2 + t1 @ W2
    wof = s2[:, None] * W_out
    bof = b_out + t2 @ W_out

    uT, itT, bu, bi = _sc_gather(user, item,
                                 user_cf_table.T, item_cf_table.T,
                                 user_bias_table.T,
                                 item_bias_table.T)
    predT = _tc_mlp(uT, itT, bu.reshape(1, B), bi.reshape(1, B),
                    W0[:D].T, W0[D:2 * D].T, W0[2 * D:].T, b0[:, None],
                    w1f.T, b1f[:, None], w2f.T, b2f[:, None],
                    wof.T, bof[:, None])
    return predT.reshape(B, 1)


# speedup vs baseline: 1.7016x; 1.0736x over previous
import functools

import jax
import jax.numpy as jnp
from jax import lax
from jax.experimental import pallas as pl
from jax.experimental.pallas import tpu as pltpu
from jax.experimental.pallas import tpu_sc as plsc

B = 16384
D = 16
EPS = 1e-3

_NC = 2
_NS = 16
_NW = _NC * _NS
_BPW = B // _NW
_G = 16
_NGRP = _BPW // _G

_MESH = plsc.VectorSubcoreMesh(core_axis_name="c", subcore_axis_name="s")


def _sc_gather(user, item, ucfT, icfT, ubT, ibT):
    @functools.partial(
        pl.kernel,
        out_type=(
            jax.ShapeDtypeStruct((D, B), jnp.float32),
            jax.ShapeDtypeStruct((D, B), jnp.float32),
            jax.ShapeDtypeStruct((B,), jnp.float32),
            jax.ShapeDtypeStruct((B,), jnp.float32),
        ),
        mesh=_MESH,
        compiler_params=pltpu.CompilerParams(needs_layout_passes=False),
        scratch_types=[
            pltpu.VMEM((_BPW,), jnp.int32),
            pltpu.VMEM((_BPW,), jnp.int32),
            pltpu.VMEM((D, _G * 128), jnp.float32),
            pltpu.VMEM((D, _G * 128), jnp.float32),
            pltpu.VMEM((D, _G * 128), jnp.float32),
            pltpu.VMEM((D, _BPW), jnp.float32),
            pltpu.VMEM((D, _BPW), jnp.float32),
            pltpu.VMEM((_BPW,), jnp.float32),
            pltpu.VMEM((_BPW,), jnp.float32),
            pltpu.SemaphoreType.DMA,
            pltpu.SemaphoreType.DMA,
            pltpu.SemaphoreType.DMA,
            pltpu.SemaphoreType.DMA,
        ],
    )
    def k(user_h, item_h, ucfT_h, icfT_h, ub_h, ib_h,
          u_out, i_out, bu_out, bi_out,
          uidx_v, iidx_v, bankA, bankB, bankC, u_v, i_v, bu_v, bi_v,
          semA, semB, semC, semD):
        wid = lax.axis_index("s") * _NC + lax.axis_index("c")
        base = wid * _BPW
        pltpu.sync_copy(user_h.at[pl.ds(base, _BPW)], uidx_v)
        pltpu.sync_copy(item_h.at[pl.ds(base, _BPW)], iidx_v)
        hb0 = pltpu.async_copy(ub_h.at[0].at[uidx_v], bu_v, semD)
        hb1 = pltpu.async_copy(ib_h.at[0].at[iidx_v], bi_v, semD)
        iota16 = lax.iota(jnp.int32, 16)

        def run_table(tab_h, idx_v, cols_v):
            def fire(g, bank, sem):
                vec = idx_v[pl.ds(g * _G, _G)]
                start_v = (vec >> 7) << 7
                for l in range(_G):
                    s = pl.multiple_of(start_v[l], 128)
                    pltpu.async_copy(
                        tab_h.at[:, pl.ds(s, 128)],
                        bank.at[:, pl.ds(l * 128, 128)],
                        sem)

            def drain(bank, sem):
                pltpu.make_async_copy(
                    tab_h.at[:, pl.ds(0, _G * 128)], bank, sem).wait()

            def extract(g, bank):
                vec = idx_v[pl.ds(g * _G, _G)]
                cols16 = (vec & 127) + iota16 * 128
                for f in range(D):
                    vals = plsc.load_gather(
                        bank, [jnp.full((16,), f, jnp.int32), cols16])
                    cols_v[f, pl.ds(g * _G, _G)] = vals

            fire(0, bankA, semA)
            fire(1, bankB, semB)

            def step(h, _):
                fire(3 * h + 2, bankC, semC)
                drain(bankA, semA)
                extract(3 * h, bankA)
                fire(3 * h + 3, bankA, semA)
                drain(bankB, semB)
                extract(3 * h + 1, bankB)
                fire(3 * h + 4, bankB, semB)
                drain(bankC, semC)
                extract(3 * h + 2, bankC)
                return ()

            lax.fori_loop(0, _NGRP // 3, step, ())
            drain(bankA, semA)
            extract(_NGRP - 2, bankA)
            drain(bankB, semB)
            extract(_NGRP - 1, bankB)

        run_table(ucfT_h, uidx_v, u_v)
        run_table(icfT_h, iidx_v, i_v)
        pltpu.sync_copy(u_v, u_out.at[:, pl.ds(base, _BPW)])
        pltpu.sync_copy(i_v, i_out.at[:, pl.ds(base, _BPW)])
        hb0.wait()
        hb1.wait()
        pltpu.sync_copy(bu_v, bu_out.at[pl.ds(base, _BPW)])
        pltpu.sync_copy(bi_v, bi_out.at[pl.ds(base, _BPW)])

    return k(user, item, ucfT, icfT, ubT, ibT)


_BM = 2048


def _mlp_body(u_ref, i_ref, bu_ref, bi_ref,
              w0aT_ref, w0bT_ref, w0cT_ref, b0_ref,
              w1T_ref, b1_ref, w2T_ref, b2_ref, woT_ref, bo_ref,
              out_ref):
    u = u_ref[...]
    it = i_ref[...]
    f32 = jnp.float32
    h = (jnp.dot(w0aT_ref[...], u, preferred_element_type=f32)
         + jnp.dot(w0bT_ref[...], it, preferred_element_type=f32)
         + jnp.dot(w0cT_ref[...], u * it, preferred_element_type=f32)
         + b0_ref[...])
    h = jnp.maximum(h, 0.0)
    h = jnp.maximum(jnp.dot(w1T_ref[...], h, preferred_element_type=f32)
                    + b1_ref[...], 0.0)
    h = jnp.maximum(jnp.dot(w2T_ref[...], h, preferred_element_type=f32)
                    + b2_ref[...], 0.0)
    out_ref[...] = (jnp.dot(woT_ref[...], h, preferred_element_type=f32)
                    + bo_ref[...] + bu_ref[...] + bi_ref[...])


def _tc_mlp(uT, itT, buR, biR,
            w0aT, w0bT, w0cT, b0c, w1T, b1c, w2T, b2c, woT, boc):
    grid = (B // _BM,)
    col_spec = pl.BlockSpec((D, _BM), lambda i: (0, i))
    row_spec = pl.BlockSpec((1, _BM), lambda i: (0, i))

    def full(a):
        return pl.BlockSpec(a.shape, lambda i: tuple(0 for _ in a.shape))

    return pl.pallas_call(
        _mlp_body,
        grid=grid,
        in_specs=[col_spec, col_spec, row_spec, row_spec,
                  full(w0aT), full(w0bT), full(w0cT), full(b0c),
                  full(w1T), full(b1c), full(w2T), full(b2c),
                  full(woT), full(boc)],
        out_specs=row_spec,
        out_shape=jax.ShapeDtypeStruct((1, B), jnp.float32),
    )(uT, itT, buR, biR,
      w0aT, w0bT, w0cT, b0c, w1T, b1c, w2T, b2c, woT, boc)


def kernel(user, item, user_cf_table, item_cf_table, user_bias_table,
           item_bias_table,
           W0, b0, gamma0, beta0, mm0, mv0,
           W1, b1, gamma1, beta1, mm1, mv1,
           W2, b2, gamma2, beta2, mm2, mv2,
           W_out, b_out):
    s0 = gamma0 * lax.rsqrt(mv0 + EPS)
    t0 = beta0 - mm0 * s0
    s1 = gamma1 * lax.rsqrt(mv1 + EPS)
    t1 = beta1 - mm1 * s1
    s2 = gamma2 * lax.rsqrt(mv2 + EPS)
    t2 = beta2 - mm2 * s2

    w1f = s0[:, None] * W1
    b1f = b1 + t0 @ W1
    w2f = s1[:, None] * W2
    b2f = b2 + t1 @ W2
    wof = s2[:, None] * W_out
    bof = b_out + t2 @ W_out

    uT, itT, bu, bi = _sc_gather(user, item,
                                 user_cf_table.T, item_cf_table.T,
                                 user_bias_table.T,
                                 item_bias_table.T)
    predT = _tc_mlp(uT, itT, bu.reshape(1, B), bi.reshape(1, B),
                    W0[:D].T, W0[D:2 * D].T, W0[2 * D:].T, b0[:, None],
                    w1f.T, b1f[:, None], w2f.T, b2f[:, None],
                    wof.T, bof[:, None])
    return predT.reshape(B, 1)
